# Initial kernel scaffold; baseline (speedup 1.0000x reference)
import jax
import jax.numpy as jnp
from jax.experimental import pallas as pl


def kernel(x, edge_index, W1, b1, W2, b2, att_w1, att_b1, att_w2):
    raise NotImplementedError("write your pallas kernel here")


# speedup vs baseline: 2.0896x; 2.0896x over previous
import dataclasses
import functools

import jax
import jax.numpy as jnp
from jax import lax
from jax.experimental import pallas as pl
from jax.experimental.pallas import tpu as pltpu
from jax.experimental.pallas import tpu_sc as plsc

N = 10000
E = 320000
D = 128
DH = 64
NPAD = 10240
EPAD = 327680
CHUNK = 128
N_TILES = 32
NSTAGE = 10240
ROWS_PER_TILE = NSTAGE // 16

_MESH = plsc.VectorSubcoreMesh(core_axis_name="c", subcore_axis_name="s")
_CP = pltpu.CompilerParams()
if "needs_layout_passes" in pltpu.CompilerParams.__dataclass_fields__:
    _CP = dataclasses.replace(_CP, needs_layout_passes=False)

_IOTA16 = lambda: lax.iota(jnp.int32, 16)


ECHUNK = 64


def _sc_edge_stats(h, src2d, dst2d, nrm2, mode):
    n_chunks = EPAD // (N_TILES * ECHUNK)
    want_dot = mode in ("both", "cos")
    want_eud = mode in ("both", "eud")
    want_np = mode in ("both", "cos")

    out_type = []
    if want_dot:
        out_type.append(jax.ShapeDtypeStruct((EPAD,), jnp.float32))
    if want_eud:
        out_type.append(jax.ShapeDtypeStruct((EPAD,), jnp.float32))
    if want_np:
        out_type.append(jax.ShapeDtypeStruct((EPAD,), jnp.float32))

    scratch = [
        pltpu.VMEM_SHARED((NSTAGE, D), jnp.float32),
        pltpu.VMEM((1, ECHUNK), jnp.int32),
        pltpu.VMEM((1, ECHUNK), jnp.int32),
        pltpu.VMEM((ECHUNK, D), jnp.float32),
        pltpu.VMEM((ECHUNK, D), jnp.float32),
        pltpu.VMEM((NPAD,), jnp.float32),
        pltpu.VMEM((ECHUNK,), jnp.float32),
        pltpu.VMEM((ECHUNK,), jnp.float32),
        pltpu.VMEM((ECHUNK,), jnp.float32),
        pltpu.SemaphoreType.DMA,
    ]

    @functools.partial(pl.kernel, out_type=out_type, mesh=_MESH,
                       compiler_params=_CP, scratch_types=scratch)
    def k(h_hbm, s_hbm, d_hbm, nrm2_hbm, *refs):
        outs = list(refs[: len(out_type)])
        sp_h, sidx, didx, abuf, bbuf, nbuf, dob, eob, npb, sem = refs[len(out_type):]
        o_dot = outs.pop(0) if want_dot else None
        o_eud = outs.pop(0) if want_eud else None
        o_np = outs.pop(0) if want_np else None

        cid = lax.axis_index("c")
        sid = lax.axis_index("s")
        wid = cid * 16 + sid

        pltpu.sync_copy(h_hbm.at[pl.ds(sid * ROWS_PER_TILE, ROWS_PER_TILE)],
                        sp_h.at[pl.ds(sid * ROWS_PER_TILE, ROWS_PER_TILE)])
        if want_np:
            pltpu.sync_copy(nrm2_hbm, nbuf)
        plsc.subcore_barrier()

        tile_row0 = wid * n_chunks

        @pl.loop(0, n_chunks)
        def _(ci):
            row = tile_row0 + ci
            pltpu.sync_copy(s_hbm.at[row], sidx)
            pltpu.sync_copy(d_hbm.at[row], didx)
            pltpu.async_copy(sp_h.at[sidx.at[0]], abuf, sem).wait()
            pltpu.async_copy(sp_h.at[didx.at[0]], bbuf, sem).wait()

            @pl.loop(0, ECHUNK // 16)
            def _(g):
                rows16 = g * 16 + _IOTA16()

                def jbody(j, carry):
                    dacc, eacc = carry
                    jv = jnp.full((16,), 0, jnp.int32) + j
                    av = plsc.load_gather(abuf, [rows16, jv])
                    bv = plsc.load_gather(bbuf, [rows16, jv])
                    if want_dot:
                        dacc = dacc + av * bv
                    if want_eud:
                        dd = av - bv + jnp.float32(1e-6)
                        eacc = eacc + dd * dd
                    return dacc, eacc

                z = jnp.zeros((16,), jnp.float32)
                dacc, eacc = pl.loop(0, D, init_carry=(z, z), unroll=8)(jbody)
                if want_dot:
                    dob[pl.ds(g * 16, 16)] = dacc
                if want_eud:
                    eob[pl.ds(g * 16, 16)] = eacc
                if want_np:
                    sv = sidx[0, pl.ds(g * 16, 16)]
                    dv = didx[0, pl.ds(g * 16, 16)]
                    na = plsc.load_gather(nbuf, [sv])
                    nb = plsc.load_gather(nbuf, [dv])
                    npb[pl.ds(g * 16, 16)] = na * nb

            e0 = row * ECHUNK
            if want_dot:
                pltpu.sync_copy(dob, o_dot.at[pl.ds(e0, ECHUNK)])
            if want_eud:
                pltpu.sync_copy(eob, o_eud.at[pl.ds(e0, ECHUNK)])
            if want_np:
                pltpu.sync_copy(npb, o_np.at[pl.ds(e0, ECHUNK)])

    return k(h, src2d, dst2d, nrm2)


def _sc_degree(ws, dst2d):
    n_w = len(ws)
    n_chunks = EPAD // (N_TILES * CHUNK)
    out_type = [jax.ShapeDtypeStruct((N_TILES, NPAD), jnp.float32)
                for _ in range(n_w)]
    scratch = (
        [pltpu.VMEM((NPAD,), jnp.float32) for _ in range(n_w)]
        + [pltpu.VMEM((1, CHUNK), jnp.int32)]
        + [pltpu.VMEM((CHUNK,), jnp.float32) for _ in range(n_w)]
    )

    @functools.partial(pl.kernel, out_type=out_type, mesh=_MESH,
                       compiler_params=_CP, scratch_types=scratch)
    def k(*refs):
        w_hbm = refs[:n_w]
        d_hbm = refs[n_w]
        outs = refs[n_w + 1: 2 * n_w + 1]
        degb = refs[2 * n_w + 1: 3 * n_w + 1]
        didx = refs[3 * n_w + 1]
        wbuf = refs[3 * n_w + 2: 4 * n_w + 2]

        cid = lax.axis_index("c")
        sid = lax.axis_index("s")
        wid = cid * 16 + sid

        @pl.loop(0, NPAD, step=16)
        def _(i):
            for b in range(n_w):
                degb[b][pl.ds(i, 16)] = jnp.zeros((16,), jnp.float32)

        tile_row0 = wid * n_chunks

        @pl.loop(0, n_chunks)
        def _(ci):
            row = tile_row0 + ci
            pltpu.sync_copy(d_hbm.at[row], didx)
            for b in range(n_w):
                pltpu.sync_copy(w_hbm[b].at[pl.ds(row * CHUNK, CHUNK)], wbuf[b])

            @pl.loop(0, CHUNK // 16)
            def _(g):
                iv = didx[0, pl.ds(g * 16, 16)]
                for b in range(n_w):
                    vv = wbuf[b][pl.ds(g * 16, 16)]
                    plsc.addupdate_scatter(degb[b], [iv], vv)

        for b in range(n_w):
            pltpu.sync_copy(degb[b], outs[b].at[wid])

    return k(*ws, dst2d)


MCHUNK = 64


def _sc_messages(hm_pad, w_list, dinv_list, src1d, dst1d):
    n_b = len(w_list)
    n_chunks = EPAD // (N_TILES * MCHUNK)

    out_type = [jax.ShapeDtypeStruct((2 * NSTAGE, D), jnp.float32)
                for _ in range(n_b)]
    scratch = (
        [pltpu.VMEM_SHARED((NSTAGE, D), jnp.float32) for _ in range(n_b)]
        + [pltpu.VMEM((MCHUNK, D), jnp.float32)]
        + [pltpu.VMEM((MCHUNK, D), jnp.float32) for _ in range(n_b)]
        + [pltpu.VMEM((NPAD,), jnp.float32) for _ in range(n_b)]
        + [pltpu.VMEM((MCHUNK,), jnp.int32)] * 2
        + [pltpu.VMEM((MCHUNK,), jnp.float32) for _ in range(n_b)]
        + [pltpu.SemaphoreType.DMA]
    )

    @functools.partial(pl.kernel, out_type=out_type, mesh=_MESH,
                       compiler_params=_CP, scratch_types=scratch)
    def k(*refs):
        i = 0
        hm_hbm = refs[i]; i += 1
        w_hbm = refs[i: i + n_b]; i += n_b
        dinv_hbm = refs[i: i + n_b]; i += n_b
        s_hbm, d_hbm = refs[i], refs[i + 1]; i += 2
        outs = refs[i: i + n_b]; i += n_b
        sp_out = refs[i: i + n_b]; i += n_b
        gbuf = refs[i]; i += 1
        sbufs = refs[i: i + n_b]; i += n_b
        dinvb = refs[i: i + n_b]; i += n_b
        sidx, didx = refs[i], refs[i + 1]; i += 2
        wbufs = refs[i: i + n_b]; i += n_b
        sem = refs[i]

        cid = lax.axis_index("c")
        sid = lax.axis_index("s")
        wid = cid * 16 + sid

        @pl.loop(0, MCHUNK)
        def _(r):
            @pl.loop(0, D, step=16)
            def _(j):
                sbufs[0].at[r][pl.ds(j, 16)] = jnp.zeros((16,), jnp.float32)

        r0 = sid * ROWS_PER_TILE
        @pl.loop(0, ROWS_PER_TILE, step=MCHUNK)
        def _(rr):
            for b in range(n_b):
                pltpu.sync_copy(sbufs[0], sp_out[b].at[pl.ds(r0 + rr, MCHUNK)])

        for b in range(n_b):
            pltpu.sync_copy(dinv_hbm[b], dinvb[b])
        plsc.subcore_barrier()

        tile_row0 = wid * n_chunks

        @pl.loop(0, n_chunks)
        def _(ci):
            e0 = (tile_row0 + ci) * MCHUNK
            pltpu.sync_copy(s_hbm.at[pl.ds(e0, MCHUNK)], sidx)
            pltpu.sync_copy(d_hbm.at[pl.ds(e0, MCHUNK)], didx)
            for b in range(n_b):
                pltpu.sync_copy(w_hbm[b].at[pl.ds(e0, MCHUNK)], wbufs[b])
            pltpu.async_copy(hm_hbm.at[sidx], gbuf, sem).wait()

            @pl.loop(0, MCHUNK // 16)
            def _(g):
                rows16 = g * 16 + _IOTA16()
                sv = sidx[pl.ds(g * 16, 16)]
                dv = didx[pl.ds(g * 16, 16)]
                norms = []
                for b in range(n_b):
                    wv = wbufs[b][pl.ds(g * 16, 16)]
                    na = plsc.load_gather(dinvb[b], [sv])
                    nb = plsc.load_gather(dinvb[b], [dv])
                    norms.append(na * wv * nb)

                def jbody(j, _):
                    jv = jnp.full((16,), 0, jnp.int32) + j
                    col = plsc.load_gather(gbuf, [rows16, jv])
                    for b in range(n_b):
                        plsc.store_scatter(sbufs[b], [rows16, jv],
                                           col * norms[b])
                    return 0

                pl.loop(0, D, init_carry=0, unroll=8)(jbody)

            for b in range(n_b):
                pltpu.sync_copy(sbufs[b], sp_out[b].at[didx], add=True)

        plsc.subcore_barrier()
        for c in range(2):
            @pl.when(cid == c)
            def _():
                for b in range(n_b):
                    pltpu.sync_copy(
                        sp_out[b].at[pl.ds(r0, ROWS_PER_TILE)],
                        outs[b].at[pl.ds(c * NSTAGE + r0, ROWS_PER_TILE)])

    return k(hm_pad, *w_list, *dinv_list, src1d, dst1d)


_NB = 10
_RB = N // _NB


def _tc_matmul_stats(h, W, with_stats):
    def body(h_ref, w_ref, hm_ref, *stat_ref):
        hb = h_ref[...]
        hm_ref[...] = jnp.dot(hb, w_ref[...],
                              preferred_element_type=jnp.float32)
        if with_stats:
            stat_ref[0][...] = jnp.broadcast_to(
                jnp.sum(hb * hb, axis=1, keepdims=True), hb.shape)

    out_shape = [jax.ShapeDtypeStruct((N, D), jnp.float32)]
    out_specs = [pl.BlockSpec((_RB, D), lambda i: (i, 0))]
    if with_stats:
        out_shape.append(jax.ShapeDtypeStruct((N, D), jnp.float32))
        out_specs.append(pl.BlockSpec((_RB, D), lambda i: (i, 0)))

    res = pl.pallas_call(
        body,
        grid=(_NB,),
        in_specs=[pl.BlockSpec((_RB, D), lambda i: (i, 0)),
                  pl.BlockSpec((D, D), lambda i: (0, 0))],
        out_specs=out_specs,
        out_shape=out_shape,
    )(h, W)
    if with_stats:
        hm, st = res
        nrm2 = jnp.concatenate(
            [st[:, 0], jnp.zeros((NPAD - N,), jnp.float32)])
        return hm, nrm2
    return res[0]


_EB = EPAD // 128 // 10
_E_ROWS = E // 128


def _tc_edge_weights(dot, eud, nprod, mode):
    want_cos = mode in ("both", "cos")
    want_eud = mode in ("both", "eud")

    def body(*refs):
        i = 0
        dot_r = eud_r = np_r = None
        if want_cos:
            dot_r = refs[i]; i += 1
        if want_eud:
            eud_r = refs[i]; i += 1
        if want_cos:
            np_r = refs[i]; i += 1
        outs = refs[i:]
        pid = pl.program_id(0)
        row0 = pid * _EB
        rows = row0 + lax.broadcasted_iota(jnp.int32, (_EB, 128), 0)
        valid = rows < _E_ROWS
        oi = 0
        if want_cos:
            wc = dot_r[...] / jnp.maximum(jnp.sqrt(np_r[...]),
                                          jnp.float32(1e-8))
            outs[oi][...] = jnp.where(valid, wc, 0.0)
            oi += 1
        if want_eud:
            we = jnp.sqrt(jnp.maximum(eud_r[...], 0.0))
            outs[oi][...] = jnp.where(valid, we, 0.0)

    ins, in_specs = [], []
    spec = pl.BlockSpec((_EB, 128), lambda i: (i, 0))
    if want_cos:
        ins.append(dot.reshape(-1, 128)); in_specs.append(spec)
    if want_eud:
        ins.append(eud.reshape(-1, 128)); in_specs.append(spec)
    if want_cos:
        ins.append(nprod.reshape(-1, 128)); in_specs.append(spec)
    n_out = int(want_cos) + int(want_eud)
    res = pl.pallas_call(
        body,
        grid=(10,),
        in_specs=in_specs,
        out_specs=[spec] * n_out,
        out_shape=[jax.ShapeDtypeStruct((EPAD // 128, 128), jnp.float32)] * n_out,
    )(*ins)
    return [r.reshape(-1) for r in res]


def _tc_dinv(parts):
    def body(p_ref, o_ref):
        s = jnp.sum(p_ref[...], axis=0, keepdims=True) + 1.0
        safe = jnp.where(s > 0, s, 1.0)
        dinv = jnp.where(s > 0, 1.0 / jnp.sqrt(safe), 0.0)
        o_ref[...] = jnp.broadcast_to(dinv, (8, p_ref.shape[1]))

    res = pl.pallas_call(
        body,
        grid=(8,),
        in_specs=[pl.BlockSpec((N_TILES, NPAD // 8), lambda i: (0, i))],
        out_specs=pl.BlockSpec((8, NPAD // 8), lambda i: (0, i)),
        out_shape=jax.ShapeDtypeStruct((8, NPAD), jnp.float32),
    )(parts)
    return res[0]


def _tc_post(agg, hm, dinv, b, relu):
    def body(a_ref, hm_ref, di_ref, b_ref, o_ref):
        di = di_ref[...]
        out = a_ref[...] + hm_ref[...] * (di * di) + b_ref[...]
        if relu:
            out = jnp.maximum(out, 0.0)
        o_ref[...] = out

    blk = pl.BlockSpec((_RB, D), lambda i: (i, 0))
    res = pl.pallas_call(
        body,
        grid=(_NB,),
        in_specs=[blk, blk,
                  pl.BlockSpec((_RB, 1), lambda i: (i, 0)),
                  pl.BlockSpec((1, D), lambda i: (0, 0))],
        out_specs=blk,
        out_shape=jax.ShapeDtypeStruct((N, D), jnp.float32),
    )(agg, hm, dinv[:N].reshape(N, 1), b.reshape(1, D))
    return res


def _tc_attention(x1, x2, aw1, ab1, aw2):
    def body(x1_ref, x2_ref, w1_ref, b1_ref, w2_ref, o_ref):
        x1b, x2b = x1_ref[...], x2_ref[...]
        t1 = jnp.tanh(jnp.dot(x1b, w1_ref[...],
                              preferred_element_type=jnp.float32) + b1_ref[...])
        t2 = jnp.tanh(jnp.dot(x2b, w1_ref[...],
                              preferred_element_type=jnp.float32) + b1_ref[...])
        s1 = jnp.dot(t1, w2_ref[...], preferred_element_type=jnp.float32)
        s2 = jnp.dot(t2, w2_ref[...], preferred_element_type=jnp.float32)
        m = jnp.maximum(s1, s2)
        e1 = jnp.exp(s1 - m)
        e2 = jnp.exp(s2 - m)
        o_ref[...] = (e1 * x1b + e2 * x2b) / (e1 + e2)

    blk = pl.BlockSpec((_RB, D), lambda i: (i, 0))
    res = pl.pallas_call(
        body,
        grid=(_NB,),
        in_specs=[blk, blk,
                  pl.BlockSpec((D, 64), lambda i: (0, 0)),
                  pl.BlockSpec((1, 64), lambda i: (0, 0)),
                  pl.BlockSpec((64, 1), lambda i: (0, 0))],
        out_specs=blk,
        out_shape=jax.ShapeDtypeStruct((N, D), jnp.float32),
    )(x1, x2, aw1, ab1.reshape(1, 64), aw2)
    return res


def _pad_rows(a):
    return jnp.pad(a, ((0, NSTAGE - N), (0, 0)))


def _conv(h_hm, w_list, src1d, dst1d, dst2d, bias, relu):
    deg_parts = _sc_degree(w_list, dst2d)
    dinvs = [_tc_dinv(p) for p in deg_parts]
    hm_pad = _pad_rows(h_hm)
    outs = []
    for b in range(len(w_list)):
        (agg,) = _sc_messages(hm_pad, [w_list[b]], [dinvs[b]],
                              src1d, dst1d)
        full = agg[:N] + agg[NSTAGE:NSTAGE + N]
        outs.append(_tc_post(full, h_hm, dinvs[b], bias, relu))
    return outs


def kernel(x, edge_index, W1, b1, W2, b2, att_w1, att_b1, att_w2):
    src = edge_index[0]
    dst = edge_index[1]
    pad = EPAD - E
    src1d = jnp.pad(src, (0, pad))
    dst1d = jnp.pad(dst, (0, pad))
    dst2d = dst1d.reshape(EPAD // CHUNK, 1, CHUNK)
    src2e = src1d.reshape(EPAD // ECHUNK, 1, ECHUNK)
    dst2e = dst1d.reshape(EPAD // ECHUNK, 1, ECHUNK)

    hm1, nrm2x = _tc_matmul_stats(x, W1, with_stats=True)
    dot, eud, nprod = _sc_edge_stats(_pad_rows(x), src2e, dst2e, nrm2x,
                                     "both")
    w_cos, w_eud = _tc_edge_weights(dot, eud, nprod, "both")
    x1, x2 = _conv(hm1, [w_cos, w_eud], src1d, dst1d, dst2d, b1, relu=True)

    hm2a, nrm2x1 = _tc_matmul_stats(x1, W2, with_stats=True)
    dot1, nprod1 = _sc_edge_stats(_pad_rows(x1), src2e, dst2e, nrm2x1, "cos")
    (w1c,) = _tc_edge_weights(dot1, None, nprod1, "cos")
    (x1o,) = _conv(hm2a, [w1c], src1d, dst1d, dst2d, b2, relu=False)

    hm2b = _tc_matmul_stats(x2, W2, with_stats=False)
    zn = jnp.zeros((NPAD,), jnp.float32)
    (eud2,) = _sc_edge_stats(_pad_rows(x2), src2e, dst2e, zn, "eud")
    (w2e,) = _tc_edge_weights(None, eud2, None, "eud")
    (x2o,) = _conv(hm2b, [w2e], src1d, dst1d, dst2d, b2, relu=False)

    return _tc_attention(x1o, x2o, att_w1, att_b1, att_w2)


# speedup vs baseline: 2.5215x; 1.2067x over previous
import dataclasses
import functools

import jax
import jax.numpy as jnp
from jax import lax
from jax.experimental import pallas as pl
from jax.experimental.pallas import tpu as pltpu
from jax.experimental.pallas import tpu_sc as plsc

N = 10000
E = 320000
D = 128
DH = 64
NPAD = 10240
EPAD = 327680
CHUNK = 128
N_TILES = 32
NSTAGE = 10240
ROWS_PER_TILE = NSTAGE // 16

_MESH = plsc.VectorSubcoreMesh(core_axis_name="c", subcore_axis_name="s")
_CP = pltpu.CompilerParams()
if "needs_layout_passes" in pltpu.CompilerParams.__dataclass_fields__:
    _CP = dataclasses.replace(_CP, needs_layout_passes=False)

_IOTA16 = lambda: lax.iota(jnp.int32, 16)


ECHUNK = 64


def _sc_edge_stats(h, src2d, dst2d, nrm2, mode):
    n_chunks = EPAD // (N_TILES * ECHUNK)
    want_dot = mode in ("both", "cos")
    want_eud = mode in ("both", "eud")
    want_np = mode in ("both", "cos")

    out_type = []
    if want_dot:
        out_type.append(jax.ShapeDtypeStruct((EPAD,), jnp.float32))
    if want_eud:
        out_type.append(jax.ShapeDtypeStruct((EPAD,), jnp.float32))
    if want_np:
        out_type.append(jax.ShapeDtypeStruct((EPAD,), jnp.float32))

    scratch = [
        pltpu.VMEM_SHARED((NSTAGE, D), jnp.float32),
        pltpu.VMEM((1, ECHUNK), jnp.int32),
        pltpu.VMEM((1, ECHUNK), jnp.int32),
        pltpu.VMEM((ECHUNK, D), jnp.float32),
        pltpu.VMEM((ECHUNK, D), jnp.float32),
        pltpu.VMEM((NPAD,), jnp.float32),
        pltpu.VMEM((ECHUNK,), jnp.float32),
        pltpu.VMEM((ECHUNK,), jnp.float32),
        pltpu.VMEM((ECHUNK,), jnp.float32),
        pltpu.SemaphoreType.DMA,
    ]

    @functools.partial(pl.kernel, out_type=out_type, mesh=_MESH,
                       compiler_params=_CP, scratch_types=scratch)
    def k(h_hbm, s_hbm, d_hbm, nrm2_hbm, *refs):
        outs = list(refs[: len(out_type)])
        sp_h, sidx, didx, abuf, bbuf, nbuf, dob, eob, npb, sem = refs[len(out_type):]
        o_dot = outs.pop(0) if want_dot else None
        o_eud = outs.pop(0) if want_eud else None
        o_np = outs.pop(0) if want_np else None

        cid = lax.axis_index("c")
        sid = lax.axis_index("s")
        wid = cid * 16 + sid

        pltpu.sync_copy(h_hbm.at[pl.ds(sid * ROWS_PER_TILE, ROWS_PER_TILE)],
                        sp_h.at[pl.ds(sid * ROWS_PER_TILE, ROWS_PER_TILE)])
        if want_np:
            pltpu.sync_copy(nrm2_hbm, nbuf)
        plsc.subcore_barrier()

        tile_row0 = wid * n_chunks

        @pl.loop(0, n_chunks)
        def _(ci):
            row = tile_row0 + ci
            pltpu.sync_copy(s_hbm.at[row], sidx)
            pltpu.sync_copy(d_hbm.at[row], didx)
            pltpu.async_copy(sp_h.at[sidx.at[0]], abuf, sem).wait()
            pltpu.async_copy(sp_h.at[didx.at[0]], bbuf, sem).wait()

            @pl.loop(0, ECHUNK // 16)
            def _(g):
                rows16 = g * 16 + _IOTA16()

                def jbody(j, carry):
                    dacc, eacc = carry
                    jv = jnp.full((16,), 0, jnp.int32) + j
                    av = plsc.load_gather(abuf, [rows16, jv])
                    bv = plsc.load_gather(bbuf, [rows16, jv])
                    if want_dot:
                        dacc = dacc + av * bv
                    if want_eud:
                        dd = av - bv + jnp.float32(1e-6)
                        eacc = eacc + dd * dd
                    return dacc, eacc

                z = jnp.zeros((16,), jnp.float32)
                dacc, eacc = pl.loop(0, D, init_carry=(z, z), unroll=8)(jbody)
                if want_dot:
                    dob[pl.ds(g * 16, 16)] = dacc
                if want_eud:
                    eob[pl.ds(g * 16, 16)] = eacc
                if want_np:
                    sv = sidx[0, pl.ds(g * 16, 16)]
                    dv = didx[0, pl.ds(g * 16, 16)]
                    na = plsc.load_gather(nbuf, [sv])
                    nb = plsc.load_gather(nbuf, [dv])
                    npb[pl.ds(g * 16, 16)] = na * nb

            e0 = row * ECHUNK
            if want_dot:
                pltpu.sync_copy(dob, o_dot.at[pl.ds(e0, ECHUNK)])
            if want_eud:
                pltpu.sync_copy(eob, o_eud.at[pl.ds(e0, ECHUNK)])
            if want_np:
                pltpu.sync_copy(npb, o_np.at[pl.ds(e0, ECHUNK)])

    return k(h, src2d, dst2d, nrm2)


def _sc_degree(ws, dst2d):
    n_w = len(ws)
    n_chunks = EPAD // (N_TILES * CHUNK)
    out_type = [jax.ShapeDtypeStruct((N_TILES, NPAD), jnp.float32)
                for _ in range(n_w)]
    scratch = (
        [pltpu.VMEM((NPAD,), jnp.float32) for _ in range(n_w)]
        + [pltpu.VMEM((1, CHUNK), jnp.int32)]
        + [pltpu.VMEM((CHUNK,), jnp.float32) for _ in range(n_w)]
    )

    @functools.partial(pl.kernel, out_type=out_type, mesh=_MESH,
                       compiler_params=_CP, scratch_types=scratch)
    def k(*refs):
        w_hbm = refs[:n_w]
        d_hbm = refs[n_w]
        outs = refs[n_w + 1: 2 * n_w + 1]
        degb = refs[2 * n_w + 1: 3 * n_w + 1]
        didx = refs[3 * n_w + 1]
        wbuf = refs[3 * n_w + 2: 4 * n_w + 2]

        cid = lax.axis_index("c")
        sid = lax.axis_index("s")
        wid = cid * 16 + sid

        @pl.loop(0, NPAD, step=16)
        def _(i):
            for b in range(n_w):
                degb[b][pl.ds(i, 16)] = jnp.zeros((16,), jnp.float32)

        tile_row0 = wid * n_chunks

        @pl.loop(0, n_chunks)
        def _(ci):
            row = tile_row0 + ci
            pltpu.sync_copy(d_hbm.at[row], didx)
            for b in range(n_w):
                pltpu.sync_copy(w_hbm[b].at[pl.ds(row * CHUNK, CHUNK)], wbuf[b])

            @pl.loop(0, CHUNK // 16)
            def _(g):
                iv = didx[0, pl.ds(g * 16, 16)]
                for b in range(n_w):
                    vv = wbuf[b][pl.ds(g * 16, 16)]
                    plsc.addupdate_scatter(degb[b], [iv], vv)

        for b in range(n_w):
            pltpu.sync_copy(degb[b], outs[b].at[wid])

    return k(*ws, dst2d)


MCHUNK = 64


def _sc_messages(hmd_pad, w, src1d, dst1d):
    n_chunks = EPAD // (N_TILES * MCHUNK)

    out_type = jax.ShapeDtypeStruct((2 * NSTAGE, D), jnp.float32)
    scratch = (
        [pltpu.VMEM_SHARED((NSTAGE, D), jnp.float32)]
        + [pltpu.VMEM((MCHUNK, D), jnp.float32)] * 2
        + [pltpu.VMEM((MCHUNK,), jnp.int32)] * 4
        + [pltpu.VMEM((MCHUNK,), jnp.float32)] * 2
        + [pltpu.SemaphoreType.DMA] * 4
    )

    @functools.partial(pl.kernel, out_type=out_type, mesh=_MESH,
                       compiler_params=_CP, scratch_types=scratch)
    def k(hm_hbm, w_hbm, s_hbm, d_hbm, out_hbm,
          sp_out, gb0, gb1, si0, si1, di0, di1, wb0, wb1,
          is0, is1, gs0, gs1):
        cid = lax.axis_index("c")
        sid = lax.axis_index("s")
        wid = cid * 16 + sid
        gb = [gb0, gb1]
        si = [si0, si1]
        di = [di0, di1]
        wb = [wb0, wb1]
        isem = [is0, is1]
        gsem = [gs0, gs1]

        @pl.loop(0, MCHUNK)
        def _(r):
            @pl.loop(0, D, step=16)
            def _(j):
                gb0.at[r][pl.ds(j, 16)] = jnp.zeros((16,), jnp.float32)

        r0 = sid * ROWS_PER_TILE
        @pl.loop(0, ROWS_PER_TILE, step=MCHUNK)
        def _(rr):
            pltpu.sync_copy(gb0, sp_out.at[pl.ds(r0 + rr, MCHUNK)])
        plsc.subcore_barrier()

        tile_row0 = wid * n_chunks

        def fire_idx(ci, p):
            e0 = (tile_row0 + ci) * MCHUNK
            pltpu.async_copy(s_hbm.at[pl.ds(e0, MCHUNK)], si[p], isem[p])
            pltpu.async_copy(d_hbm.at[pl.ds(e0, MCHUNK)], di[p], isem[p])
            pltpu.async_copy(w_hbm.at[pl.ds(e0, MCHUNK)], wb[p], isem[p])

        def wait_idx(ci, p):
            e0 = (tile_row0 + ci) * MCHUNK
            pltpu.make_async_copy(s_hbm.at[pl.ds(e0, MCHUNK)], si[p],
                                  isem[p]).wait()
            pltpu.make_async_copy(d_hbm.at[pl.ds(e0, MCHUNK)], di[p],
                                  isem[p]).wait()
            pltpu.make_async_copy(w_hbm.at[pl.ds(e0, MCHUNK)], wb[p],
                                  isem[p]).wait()

        def fire_gather(p):
            pltpu.async_copy(hm_hbm.at[si[p]], gb[p], gsem[p])

        def wait_gather(p):
            pltpu.make_async_copy(hm_hbm.at[si[p]], gb[p], gsem[p]).wait()

        def compute_scatter(p):
            @pl.loop(0, MCHUNK // 16)
            def _(g):
                rows16 = g * 16 + _IOTA16()
                wv = wb[p][pl.ds(g * 16, 16)]

                def jbody(j, _):
                    jv = jnp.full((16,), 0, jnp.int32) + j
                    col = plsc.load_gather(gb[p], [rows16, jv])
                    plsc.store_scatter(gb[p], [rows16, jv], col * wv)
                    return 0

                pl.loop(0, D, init_carry=0, unroll=8)(jbody)

            pltpu.sync_copy(gb[p], sp_out.at[di[p]], add=True)

        fire_idx(0, 0)
        wait_idx(0, 0)
        fire_gather(0)

        @pl.loop(0, n_chunks, step=2)
        def _(ci):
            fire_idx(ci + 1, 1)
            wait_gather(0)
            wait_idx(ci + 1, 1)
            fire_gather(1)
            compute_scatter(0)

            @pl.when(ci + 2 < n_chunks)
            def _():
                fire_idx(ci + 2, 0)
            wait_gather(1)

            @pl.when(ci + 2 < n_chunks)
            def _():
                wait_idx(ci + 2, 0)
                fire_gather(0)
            compute_scatter(1)

        plsc.subcore_barrier()
        for c in range(2):
            @pl.when(cid == c)
            def _():
                pltpu.sync_copy(
                    sp_out.at[pl.ds(r0, ROWS_PER_TILE)],
                    out_hbm.at[pl.ds(c * NSTAGE + r0, ROWS_PER_TILE)])

    return k(hmd_pad, w, src1d, dst1d)


_NB = 10
_RB = N // _NB


def _tc_matmul_stats(h, W, with_stats):
    def body(h_ref, w_ref, hm_ref, *stat_ref):
        hb = h_ref[...]
        hm_ref[...] = jnp.dot(hb, w_ref[...],
                              preferred_element_type=jnp.float32)
        if with_stats:
            stat_ref[0][...] = jnp.broadcast_to(
                jnp.sum(hb * hb, axis=1, keepdims=True), hb.shape)

    out_shape = [jax.ShapeDtypeStruct((N, D), jnp.float32)]
    out_specs = [pl.BlockSpec((_RB, D), lambda i: (i, 0))]
    if with_stats:
        out_shape.append(jax.ShapeDtypeStruct((N, D), jnp.float32))
        out_specs.append(pl.BlockSpec((_RB, D), lambda i: (i, 0)))

    res = pl.pallas_call(
        body,
        grid=(_NB,),
        in_specs=[pl.BlockSpec((_RB, D), lambda i: (i, 0)),
                  pl.BlockSpec((D, D), lambda i: (0, 0))],
        out_specs=out_specs,
        out_shape=out_shape,
    )(h, W)
    if with_stats:
        hm, st = res
        nrm2 = jnp.concatenate(
            [st[:, 0], jnp.zeros((NPAD - N,), jnp.float32)])
        return hm, nrm2
    return res[0]


_EB = EPAD // 128 // 10
_E_ROWS = E // 128


def _tc_edge_weights(dot, eud, nprod, mode):
    want_cos = mode in ("both", "cos")
    want_eud = mode in ("both", "eud")

    def body(*refs):
        i = 0
        dot_r = eud_r = np_r = None
        if want_cos:
            dot_r = refs[i]; i += 1
        if want_eud:
            eud_r = refs[i]; i += 1
        if want_cos:
            np_r = refs[i]; i += 1
        outs = refs[i:]
        pid = pl.program_id(0)
        row0 = pid * _EB
        rows = row0 + lax.broadcasted_iota(jnp.int32, (_EB, 128), 0)
        valid = rows < _E_ROWS
        oi = 0
        if want_cos:
            wc = dot_r[...] / jnp.maximum(jnp.sqrt(np_r[...]),
                                          jnp.float32(1e-8))
            outs[oi][...] = jnp.where(valid, wc, 0.0)
            oi += 1
        if want_eud:
            we = jnp.sqrt(jnp.maximum(eud_r[...], 0.0))
            outs[oi][...] = jnp.where(valid, we, 0.0)

    ins, in_specs = [], []
    spec = pl.BlockSpec((_EB, 128), lambda i: (i, 0))
    if want_cos:
        ins.append(dot.reshape(-1, 128)); in_specs.append(spec)
    if want_eud:
        ins.append(eud.reshape(-1, 128)); in_specs.append(spec)
    if want_cos:
        ins.append(nprod.reshape(-1, 128)); in_specs.append(spec)
    n_out = int(want_cos) + int(want_eud)
    res = pl.pallas_call(
        body,
        grid=(10,),
        in_specs=in_specs,
        out_specs=[spec] * n_out,
        out_shape=[jax.ShapeDtypeStruct((EPAD // 128, 128), jnp.float32)] * n_out,
    )(*ins)
    return [r.reshape(-1) for r in res]


def _tc_dinv(parts):
    def body(p_ref, o_ref):
        s = jnp.sum(p_ref[...], axis=0, keepdims=True) + 1.0
        safe = jnp.where(s > 0, s, 1.0)
        dinv = jnp.where(s > 0, 1.0 / jnp.sqrt(safe), 0.0)
        o_ref[...] = jnp.broadcast_to(dinv, (8, p_ref.shape[1]))

    res = pl.pallas_call(
        body,
        grid=(8,),
        in_specs=[pl.BlockSpec((N_TILES, NPAD // 8), lambda i: (0, i))],
        out_specs=pl.BlockSpec((8, NPAD // 8), lambda i: (0, i)),
        out_shape=jax.ShapeDtypeStruct((8, NPAD), jnp.float32),
    )(parts)
    return res[0]


def _tc_scale_rows(hm, dinv):
    def body(hm_ref, di_ref, o_ref):
        o_ref[...] = hm_ref[...] * di_ref[...]

    blk = pl.BlockSpec((_RB, D), lambda i: (i, 0))
    return pl.pallas_call(
        body,
        grid=(_NB,),
        in_specs=[blk, pl.BlockSpec((_RB, 1), lambda i: (i, 0))],
        out_specs=blk,
        out_shape=jax.ShapeDtypeStruct((N, D), jnp.float32),
    )(hm, dinv[:N].reshape(N, 1))


def _tc_post(agg0, agg1, hmd, dinv, b, relu):
    def body(a0_ref, a1_ref, hmd_ref, di_ref, b_ref, o_ref):
        out = ((a0_ref[...] + a1_ref[...] + hmd_ref[...]) * di_ref[...]
               + b_ref[...])
        if relu:
            out = jnp.maximum(out, 0.0)
        o_ref[...] = out

    blk = pl.BlockSpec((_RB, D), lambda i: (i, 0))
    res = pl.pallas_call(
        body,
        grid=(_NB,),
        in_specs=[blk, blk, blk,
                  pl.BlockSpec((_RB, 1), lambda i: (i, 0)),
                  pl.BlockSpec((1, D), lambda i: (0, 0))],
        out_specs=blk,
        out_shape=jax.ShapeDtypeStruct((N, D), jnp.float32),
    )(agg0, agg1, hmd, dinv[:N].reshape(N, 1), b.reshape(1, D))
    return res


def _tc_attention(x1, x2, aw1, ab1, aw2):
    def body(x1_ref, x2_ref, w1_ref, b1_ref, w2_ref, o_ref):
        x1b, x2b = x1_ref[...], x2_ref[...]
        t1 = jnp.tanh(jnp.dot(x1b, w1_ref[...],
                              preferred_element_type=jnp.float32) + b1_ref[...])
        t2 = jnp.tanh(jnp.dot(x2b, w1_ref[...],
                              preferred_element_type=jnp.float32) + b1_ref[...])
        s1 = jnp.dot(t1, w2_ref[...], preferred_element_type=jnp.float32)
        s2 = jnp.dot(t2, w2_ref[...], preferred_element_type=jnp.float32)
        m = jnp.maximum(s1, s2)
        e1 = jnp.exp(s1 - m)
        e2 = jnp.exp(s2 - m)
        o_ref[...] = (e1 * x1b + e2 * x2b) / (e1 + e2)

    blk = pl.BlockSpec((_RB, D), lambda i: (i, 0))
    res = pl.pallas_call(
        body,
        grid=(_NB,),
        in_specs=[blk, blk,
                  pl.BlockSpec((D, 64), lambda i: (0, 0)),
                  pl.BlockSpec((1, 64), lambda i: (0, 0)),
                  pl.BlockSpec((64, 1), lambda i: (0, 0))],
        out_specs=blk,
        out_shape=jax.ShapeDtypeStruct((N, D), jnp.float32),
    )(x1, x2, aw1, ab1.reshape(1, 64), aw2)
    return res


def _pad_rows(a):
    return jnp.pad(a, ((0, NSTAGE - N), (0, 0)))


def _conv(h_hm, w_list, src1d, dst1d, dst2d, bias, relu):
    deg_parts = _sc_degree(w_list, dst2d)
    dinvs = [_tc_dinv(p) for p in deg_parts]
    outs = []
    for b in range(len(w_list)):
        hmd = _tc_scale_rows(h_hm, dinvs[b])
        agg = _sc_messages(_pad_rows(hmd), w_list[b], src1d, dst1d)
        outs.append(_tc_post(agg[:N], agg[NSTAGE:NSTAGE + N], hmd,
                             dinvs[b], bias, relu))
    return outs


def kernel(x, edge_index, W1, b1, W2, b2, att_w1, att_b1, att_w2):
    src = edge_index[0]
    dst = edge_index[1]
    pad = EPAD - E
    src1d = jnp.pad(src, (0, pad))
    dst1d = jnp.pad(dst, (0, pad))
    dst2d = dst1d.reshape(EPAD // CHUNK, 1, CHUNK)
    src2e = src1d.reshape(EPAD // ECHUNK, 1, ECHUNK)
    dst2e = dst1d.reshape(EPAD // ECHUNK, 1, ECHUNK)

    hm1, nrm2x = _tc_matmul_stats(x, W1, with_stats=True)
    dot, eud, nprod = _sc_edge_stats(_pad_rows(x), src2e, dst2e, nrm2x,
                                     "both")
    w_cos, w_eud = _tc_edge_weights(dot, eud, nprod, "both")
    x1, x2 = _conv(hm1, [w_cos, w_eud], src1d, dst1d, dst2d, b1, relu=True)

    hm2a, nrm2x1 = _tc_matmul_stats(x1, W2, with_stats=True)
    dot1, nprod1 = _sc_edge_stats(_pad_rows(x1), src2e, dst2e, nrm2x1, "cos")
    (w1c,) = _tc_edge_weights(dot1, None, nprod1, "cos")
    (x1o,) = _conv(hm2a, [w1c], src1d, dst1d, dst2d, b2, relu=False)

    hm2b = _tc_matmul_stats(x2, W2, with_stats=False)
    zn = jnp.zeros((NPAD,), jnp.float32)
    (eud2,) = _sc_edge_stats(_pad_rows(x2), src2e, dst2e, zn, "eud")
    (w2e,) = _tc_edge_weights(None, eud2, None, "eud")
    (x2o,) = _conv(hm2b, [w2e], src1d, dst1d, dst2d, b2, relu=False)

    return _tc_attention(x1o, x2o, att_w1, att_b1, att_w2)


# speedup vs baseline: 2.5619x; 1.0161x over previous
import dataclasses
import functools

import jax
import jax.numpy as jnp
from jax import lax
from jax.experimental import pallas as pl
from jax.experimental.pallas import tpu as pltpu
from jax.experimental.pallas import tpu_sc as plsc

N = 10000
E = 320000
D = 128
DH = 64
NPAD = 10240
EPAD = 327680
CHUNK = 128
N_TILES = 32
NSTAGE = 10240
ROWS_PER_TILE = NSTAGE // 16

_MESH = plsc.VectorSubcoreMesh(core_axis_name="c", subcore_axis_name="s")
_CP = pltpu.CompilerParams()
if "needs_layout_passes" in pltpu.CompilerParams.__dataclass_fields__:
    _CP = dataclasses.replace(_CP, needs_layout_passes=False)

_IOTA16 = lambda: lax.iota(jnp.int32, 16)


ECHUNK = 64


def _sc_edge_stats(h, src2d, dst2d, nrm2, mode):
    n_chunks = EPAD // (N_TILES * ECHUNK)
    want_dot = mode in ("both", "cos")
    want_eud = mode in ("both", "eud")
    want_np = mode in ("both", "cos")

    out_type = []
    if want_dot:
        out_type.append(jax.ShapeDtypeStruct((EPAD,), jnp.float32))
    if want_eud:
        out_type.append(jax.ShapeDtypeStruct((EPAD,), jnp.float32))
    if want_np:
        out_type.append(jax.ShapeDtypeStruct((EPAD,), jnp.float32))

    scratch = [
        pltpu.VMEM_SHARED((NSTAGE, D), jnp.float32),
        pltpu.VMEM((1, ECHUNK), jnp.int32),
        pltpu.VMEM((1, ECHUNK), jnp.int32),
        pltpu.VMEM((ECHUNK, D), jnp.float32),
        pltpu.VMEM((ECHUNK, D), jnp.float32),
        pltpu.VMEM((NPAD,), jnp.float32),
        pltpu.VMEM((ECHUNK,), jnp.float32),
        pltpu.VMEM((ECHUNK,), jnp.float32),
        pltpu.VMEM((ECHUNK,), jnp.float32),
        pltpu.SemaphoreType.DMA,
    ]

    @functools.partial(pl.kernel, out_type=out_type, mesh=_MESH,
                       compiler_params=_CP, scratch_types=scratch)
    def k(h_hbm, s_hbm, d_hbm, nrm2_hbm, *refs):
        outs = list(refs[: len(out_type)])
        sp_h, sidx, didx, abuf, bbuf, nbuf, dob, eob, npb, sem = refs[len(out_type):]
        o_dot = outs.pop(0) if want_dot else None
        o_eud = outs.pop(0) if want_eud else None
        o_np = outs.pop(0) if want_np else None

        cid = lax.axis_index("c")
        sid = lax.axis_index("s")
        wid = cid * 16 + sid

        pltpu.sync_copy(h_hbm.at[pl.ds(sid * ROWS_PER_TILE, ROWS_PER_TILE)],
                        sp_h.at[pl.ds(sid * ROWS_PER_TILE, ROWS_PER_TILE)])
        if want_np:
            pltpu.sync_copy(nrm2_hbm, nbuf)
        plsc.subcore_barrier()

        tile_row0 = wid * n_chunks

        @pl.loop(0, n_chunks)
        def _(ci):
            row = tile_row0 + ci
            pltpu.sync_copy(s_hbm.at[row], sidx)
            pltpu.sync_copy(d_hbm.at[row], didx)
            pltpu.async_copy(sp_h.at[sidx.at[0]], abuf, sem).wait()
            pltpu.async_copy(sp_h.at[didx.at[0]], bbuf, sem).wait()

            @pl.loop(0, ECHUNK // 16)
            def _(g):
                rows16 = g * 16 + _IOTA16()

                def jbody(j, carry):
                    dacc, eacc = carry
                    jv = jnp.full((16,), 0, jnp.int32) + j
                    av = plsc.load_gather(abuf, [rows16, jv])
                    bv = plsc.load_gather(bbuf, [rows16, jv])
                    if want_dot:
                        dacc = dacc + av * bv
                    if want_eud:
                        dd = av - bv + jnp.float32(1e-6)
                        eacc = eacc + dd * dd
                    return dacc, eacc

                z = jnp.zeros((16,), jnp.float32)
                dacc, eacc = pl.loop(0, D, init_carry=(z, z), unroll=8)(jbody)
                if want_dot:
                    dob[pl.ds(g * 16, 16)] = dacc
                if want_eud:
                    eob[pl.ds(g * 16, 16)] = eacc
                if want_np:
                    sv = sidx[0, pl.ds(g * 16, 16)]
                    dv = didx[0, pl.ds(g * 16, 16)]
                    na = plsc.load_gather(nbuf, [sv])
                    nb = plsc.load_gather(nbuf, [dv])
                    npb[pl.ds(g * 16, 16)] = na * nb

            e0 = row * ECHUNK
            if want_dot:
                pltpu.sync_copy(dob, o_dot.at[pl.ds(e0, ECHUNK)])
            if want_eud:
                pltpu.sync_copy(eob, o_eud.at[pl.ds(e0, ECHUNK)])
            if want_np:
                pltpu.sync_copy(npb, o_np.at[pl.ds(e0, ECHUNK)])

    return k(h, src2d, dst2d, nrm2)


def _sc_degree(ws, dst2d):
    n_w = len(ws)
    n_chunks = EPAD // (N_TILES * CHUNK)
    out_type = [jax.ShapeDtypeStruct((N_TILES, NPAD), jnp.float32)
                for _ in range(n_w)]
    scratch = (
        [pltpu.VMEM((NPAD,), jnp.float32) for _ in range(n_w)]
        + [pltpu.VMEM((1, CHUNK), jnp.int32)]
        + [pltpu.VMEM((CHUNK,), jnp.float32) for _ in range(n_w)]
    )

    @functools.partial(pl.kernel, out_type=out_type, mesh=_MESH,
                       compiler_params=_CP, scratch_types=scratch)
    def k(*refs):
        w_hbm = refs[:n_w]
        d_hbm = refs[n_w]
        outs = refs[n_w + 1: 2 * n_w + 1]
        degb = refs[2 * n_w + 1: 3 * n_w + 1]
        didx = refs[3 * n_w + 1]
        wbuf = refs[3 * n_w + 2: 4 * n_w + 2]

        cid = lax.axis_index("c")
        sid = lax.axis_index("s")
        wid = cid * 16 + sid

        @pl.loop(0, NPAD, step=16)
        def _(i):
            for b in range(n_w):
                degb[b][pl.ds(i, 16)] = jnp.zeros((16,), jnp.float32)

        tile_row0 = wid * n_chunks

        @pl.loop(0, n_chunks)
        def _(ci):
            row = tile_row0 + ci
            pltpu.sync_copy(d_hbm.at[row], didx)
            for b in range(n_w):
                pltpu.sync_copy(w_hbm[b].at[pl.ds(row * CHUNK, CHUNK)], wbuf[b])

            @pl.loop(0, CHUNK // 16)
            def _(g):
                iv = didx[0, pl.ds(g * 16, 16)]
                for b in range(n_w):
                    vv = wbuf[b][pl.ds(g * 16, 16)]
                    plsc.addupdate_scatter(degb[b], [iv], vv)

        for b in range(n_w):
            pltpu.sync_copy(degb[b], outs[b].at[wid])

    return k(*ws, dst2d)


MCHUNK = 128
MSB = 16


def _sc_messages(hmd_pad, edata):
    n_chunks = EPAD // (N_TILES * MCHUNK)
    n_sb = n_chunks // MSB

    out_type = jax.ShapeDtypeStruct((2 * NSTAGE, D), jnp.float32)
    scratch = (
        [pltpu.VMEM_SHARED((NSTAGE, D), jnp.float32)]
        + [pltpu.VMEM((MCHUNK, D), jnp.float32)] * 2
        + [pltpu.VMEM((MSB, 384), jnp.int32)]
        + [pltpu.VMEM((MCHUNK,), jnp.int32)] * 4
        + [pltpu.SemaphoreType.DMA] * 2
    )

    @functools.partial(pl.kernel, out_type=out_type, mesh=_MESH,
                       compiler_params=_CP, scratch_types=scratch)
    def k(hm_hbm, ed_hbm, out_hbm,
          sp_out, gb0, gb1, ebuf, si0, si1, di0, di1, gs0, gs1):
        cid = lax.axis_index("c")
        sid = lax.axis_index("s")
        wid = cid * 16 + sid
        gb = [gb0, gb1]
        si = [si0, si1]
        di = [di0, di1]
        gsem = [gs0, gs1]

        @pl.loop(0, MCHUNK)
        def _(r):
            @pl.loop(0, D, step=16)
            def _(j):
                gb0.at[r][pl.ds(j, 16)] = jnp.zeros((16,), jnp.float32)

        r0 = sid * ROWS_PER_TILE
        @pl.loop(0, ROWS_PER_TILE, step=MCHUNK)
        def _(rr):
            pltpu.sync_copy(gb0, sp_out.at[pl.ds(r0 + rr, MCHUNK)])
        plsc.subcore_barrier()

        tile_c0 = wid * n_chunks

        def build_idx(lc, p):
            @pl.loop(0, MCHUNK, step=16)
            def _(j):
                si[p][pl.ds(j, 16)] = ebuf[lc, pl.ds(j, 16)]
                di[p][pl.ds(j, 16)] = ebuf[lc, pl.ds(MCHUNK + j, 16)]

        def fire_gather(p):
            pltpu.async_copy(hm_hbm.at[si[p]], gb[p], gsem[p])

        def wait_gather(p):
            pltpu.make_async_copy(hm_hbm.at[si[p]], gb[p], gsem[p]).wait()

        def compute_scatter(lc, p):
            @pl.loop(0, MCHUNK // 16)
            def _(g):
                rows16 = g * 16 + _IOTA16()
                wv = plsc.bitcast(ebuf[lc, pl.ds(2 * MCHUNK + g * 16, 16)],
                                  jnp.float32)

                def jbody(j, _):
                    jv = jnp.full((16,), 0, jnp.int32) + j
                    col = plsc.load_gather(gb[p], [rows16, jv])
                    plsc.store_scatter(gb[p], [rows16, jv], col * wv)
                    return 0

                pl.loop(0, D, init_carry=0, unroll=8)(jbody)

            pltpu.sync_copy(gb[p], sp_out.at[di[p]], add=True)

        @pl.loop(0, n_sb)
        def _(sb):
            pltpu.sync_copy(ed_hbm.at[pl.ds(tile_c0 + sb * MSB, MSB)], ebuf)
            build_idx(0, 0)
            fire_gather(0)

            @pl.loop(0, MSB, step=2)
            def _(lc):
                build_idx(lc + 1, 1)
                wait_gather(0)
                fire_gather(1)
                compute_scatter(lc, 0)

                @pl.when(lc + 2 < MSB)
                def _():
                    build_idx(lc + 2, 0)
                    fire_gather(0)
                wait_gather(1)
                compute_scatter(lc + 1, 1)

        plsc.subcore_barrier()
        for c in range(2):
            @pl.when(cid == c)
            def _():
                pltpu.sync_copy(
                    sp_out.at[pl.ds(r0, ROWS_PER_TILE)],
                    out_hbm.at[pl.ds(c * NSTAGE + r0, ROWS_PER_TILE)])

    return k(hmd_pad, edata)


_NB = 10
_RB = N // _NB


def _tc_matmul_stats(h, W, with_stats):
    def body(h_ref, w_ref, hm_ref, *stat_ref):
        hb = h_ref[...]
        hm_ref[...] = jnp.dot(hb, w_ref[...],
                              preferred_element_type=jnp.float32)
        if with_stats:
            stat_ref[0][...] = jnp.broadcast_to(
                jnp.sum(hb * hb, axis=1, keepdims=True), hb.shape)

    out_shape = [jax.ShapeDtypeStruct((N, D), jnp.float32)]
    out_specs = [pl.BlockSpec((_RB, D), lambda i: (i, 0))]
    if with_stats:
        out_shape.append(jax.ShapeDtypeStruct((N, D), jnp.float32))
        out_specs.append(pl.BlockSpec((_RB, D), lambda i: (i, 0)))

    res = pl.pallas_call(
        body,
        grid=(_NB,),
        in_specs=[pl.BlockSpec((_RB, D), lambda i: (i, 0)),
                  pl.BlockSpec((D, D), lambda i: (0, 0))],
        out_specs=out_specs,
        out_shape=out_shape,
    )(h, W)
    if with_stats:
        hm, st = res
        nrm2 = jnp.concatenate(
            [st[:, 0], jnp.zeros((NPAD - N,), jnp.float32)])
        return hm, nrm2
    return res[0]


_EB = EPAD // 128 // 10
_E_ROWS = E // 128


def _tc_edge_weights(dot, eud, nprod, mode):
    want_cos = mode in ("both", "cos")
    want_eud = mode in ("both", "eud")

    def body(*refs):
        i = 0
        dot_r = eud_r = np_r = None
        if want_cos:
            dot_r = refs[i]; i += 1
        if want_eud:
            eud_r = refs[i]; i += 1
        if want_cos:
            np_r = refs[i]; i += 1
        outs = refs[i:]
        pid = pl.program_id(0)
        row0 = pid * _EB
        rows = row0 + lax.broadcasted_iota(jnp.int32, (_EB, 128), 0)
        valid = rows < _E_ROWS
        oi = 0
        if want_cos:
            wc = dot_r[...] / jnp.maximum(jnp.sqrt(np_r[...]),
                                          jnp.float32(1e-8))
            outs[oi][...] = jnp.where(valid, wc, 0.0)
            oi += 1
        if want_eud:
            we = jnp.sqrt(jnp.maximum(eud_r[...], 0.0))
            outs[oi][...] = jnp.where(valid, we, 0.0)

    ins, in_specs = [], []
    spec = pl.BlockSpec((_EB, 128), lambda i: (i, 0))
    if want_cos:
        ins.append(dot.reshape(-1, 128)); in_specs.append(spec)
    if want_eud:
        ins.append(eud.reshape(-1, 128)); in_specs.append(spec)
    if want_cos:
        ins.append(nprod.reshape(-1, 128)); in_specs.append(spec)
    n_out = int(want_cos) + int(want_eud)
    res = pl.pallas_call(
        body,
        grid=(10,),
        in_specs=in_specs,
        out_specs=[spec] * n_out,
        out_shape=[jax.ShapeDtypeStruct((EPAD // 128, 128), jnp.float32)] * n_out,
    )(*ins)
    return [r.reshape(-1) for r in res]


def _tc_dinv(parts):
    def body(p_ref, o_ref):
        s = jnp.sum(p_ref[...], axis=0, keepdims=True) + 1.0
        safe = jnp.where(s > 0, s, 1.0)
        dinv = jnp.where(s > 0, 1.0 / jnp.sqrt(safe), 0.0)
        o_ref[...] = jnp.broadcast_to(dinv, (8, p_ref.shape[1]))

    res = pl.pallas_call(
        body,
        grid=(8,),
        in_specs=[pl.BlockSpec((N_TILES, NPAD // 8), lambda i: (0, i))],
        out_specs=pl.BlockSpec((8, NPAD // 8), lambda i: (0, i)),
        out_shape=jax.ShapeDtypeStruct((8, NPAD), jnp.float32),
    )(parts)
    return res[0]


def _tc_scale_rows(hm, dinv):
    def body(hm_ref, di_ref, o_ref):
        o_ref[...] = hm_ref[...] * di_ref[...]

    blk = pl.BlockSpec((_RB, D), lambda i: (i, 0))
    return pl.pallas_call(
        body,
        grid=(_NB,),
        in_specs=[blk, pl.BlockSpec((_RB, 1), lambda i: (i, 0))],
        out_specs=blk,
        out_shape=jax.ShapeDtypeStruct((N, D), jnp.float32),
    )(hm, dinv[:N].reshape(N, 1))


def _tc_post(agg0, agg1, hmd, dinv, b, relu):
    def body(a0_ref, a1_ref, hmd_ref, di_ref, b_ref, o_ref):
        out = ((a0_ref[...] + a1_ref[...] + hmd_ref[...]) * di_ref[...]
               + b_ref[...])
        if relu:
            out = jnp.maximum(out, 0.0)
        o_ref[...] = out

    blk = pl.BlockSpec((_RB, D), lambda i: (i, 0))
    res = pl.pallas_call(
        body,
        grid=(_NB,),
        in_specs=[blk, blk, blk,
                  pl.BlockSpec((_RB, 1), lambda i: (i, 0)),
                  pl.BlockSpec((1, D), lambda i: (0, 0))],
        out_specs=blk,
        out_shape=jax.ShapeDtypeStruct((N, D), jnp.float32),
    )(agg0, agg1, hmd, dinv[:N].reshape(N, 1), b.reshape(1, D))
    return res


def _tc_attention(x1, x2, aw1, ab1, aw2):
    def body(x1_ref, x2_ref, w1_ref, b1_ref, w2_ref, o_ref):
        x1b, x2b = x1_ref[...], x2_ref[...]
        t1 = jnp.tanh(jnp.dot(x1b, w1_ref[...],
                              preferred_element_type=jnp.float32) + b1_ref[...])
        t2 = jnp.tanh(jnp.dot(x2b, w1_ref[...],
                              preferred_element_type=jnp.float32) + b1_ref[...])
        s1 = jnp.dot(t1, w2_ref[...], preferred_element_type=jnp.float32)
        s2 = jnp.dot(t2, w2_ref[...], preferred_element_type=jnp.float32)
        m = jnp.maximum(s1, s2)
        e1 = jnp.exp(s1 - m)
        e2 = jnp.exp(s2 - m)
        o_ref[...] = (e1 * x1b + e2 * x2b) / (e1 + e2)

    blk = pl.BlockSpec((_RB, D), lambda i: (i, 0))
    res = pl.pallas_call(
        body,
        grid=(_NB,),
        in_specs=[blk, blk,
                  pl.BlockSpec((D, 64), lambda i: (0, 0)),
                  pl.BlockSpec((1, 64), lambda i: (0, 0)),
                  pl.BlockSpec((64, 1), lambda i: (0, 0))],
        out_specs=blk,
        out_shape=jax.ShapeDtypeStruct((N, D), jnp.float32),
    )(x1, x2, aw1, ab1.reshape(1, 64), aw2)
    return res


def _pad_rows(a):
    return jnp.pad(a, ((0, NSTAGE - N), (0, 0)))


def _conv(h_hm, w_list, sd_m, dst2d, bias, relu):
    deg_parts = _sc_degree(w_list, dst2d)
    dinvs = [_tc_dinv(p) for p in deg_parts]
    outs = []
    for b in range(len(w_list)):
        hmd = _tc_scale_rows(h_hm, dinvs[b])
        wm = lax.bitcast_convert_type(w_list[b], jnp.int32).reshape(-1, 128)
        edata = jnp.concatenate([sd_m, wm], axis=1)
        agg = _sc_messages(_pad_rows(hmd), edata)
        outs.append(_tc_post(agg[:N], agg[NSTAGE:NSTAGE + N], hmd,
                             dinvs[b], bias, relu))
    return outs


def kernel(x, edge_index, W1, b1, W2, b2, att_w1, att_b1, att_w2):
    src = edge_index[0]
    dst = edge_index[1]
    pad = EPAD - E
    src1d = jnp.pad(src, (0, pad))
    dst1d = jnp.pad(dst, (0, pad))
    dst2d = dst1d.reshape(EPAD // CHUNK, 1, CHUNK)
    src2e = src1d.reshape(EPAD // ECHUNK, 1, ECHUNK)
    dst2e = dst1d.reshape(EPAD // ECHUNK, 1, ECHUNK)
    sd_m = jnp.concatenate([src1d.reshape(-1, 128),
                            dst1d.reshape(-1, 128)], axis=1)

    hm1, nrm2x = _tc_matmul_stats(x, W1, with_stats=True)
    dot, eud, nprod = _sc_edge_stats(_pad_rows(x), src2e, dst2e, nrm2x,
                                     "both")
    w_cos, w_eud = _tc_edge_weights(dot, eud, nprod, "both")
    x1, x2 = _conv(hm1, [w_cos, w_eud], sd_m, dst2d, b1, relu=True)

    hm2a, nrm2x1 = _tc_matmul_stats(x1, W2, with_stats=True)
    dot1, nprod1 = _sc_edge_stats(_pad_rows(x1), src2e, dst2e, nrm2x1, "cos")
    (w1c,) = _tc_edge_weights(dot1, None, nprod1, "cos")
    (x1o,) = _conv(hm2a, [w1c], sd_m, dst2d, b2, relu=False)

    hm2b = _tc_matmul_stats(x2, W2, with_stats=False)
    zn = jnp.zeros((NPAD,), jnp.float32)
    (eud2,) = _sc_edge_stats(_pad_rows(x2), src2e, dst2e, zn, "eud")
    (w2e,) = _tc_edge_weights(None, eud2, None, "eud")
    (x2o,) = _conv(hm2b, [w2e], sd_m, dst2d, b2, relu=False)

    return _tc_attention(x1o, x2o, att_w1, att_b1, att_w2)


# speedup vs baseline: 2.6756x; 1.0444x over previous
import dataclasses
import functools

import jax
import jax.numpy as jnp
from jax import lax
from jax.experimental import pallas as pl
from jax.experimental.pallas import tpu as pltpu
from jax.experimental.pallas import tpu_sc as plsc

N = 10000
E = 320000
D = 128
DH = 64
NPAD = 10240
EPAD = 327680
CHUNK = 128
N_TILES = 32
NSTAGE = 10240
ROWS_PER_TILE = NSTAGE // 16

_MESH = plsc.VectorSubcoreMesh(core_axis_name="c", subcore_axis_name="s")
_CP = pltpu.CompilerParams()
if "needs_layout_passes" in pltpu.CompilerParams.__dataclass_fields__:
    _CP = dataclasses.replace(_CP, needs_layout_passes=False)

_IOTA16 = lambda: lax.iota(jnp.int32, 16)


ECHUNK = 64
ESB = 16


def _sc_edge_stats(h, sd_e, nrm2, mode):
    n_chunks = EPAD // (N_TILES * ECHUNK)
    n_sb = n_chunks // ESB
    SBE = ESB * ECHUNK
    want_dot = mode in ("both", "cos")
    want_eud = mode in ("both", "eud")
    want_np = mode in ("both", "cos")

    out_type = [jax.ShapeDtypeStruct((EPAD,), jnp.float32)
                for _ in range(int(want_dot) + int(want_eud) + int(want_np))]

    scratch = [
        pltpu.VMEM_SHARED((NSTAGE, D), jnp.float32),
        pltpu.VMEM((ESB, 2 * ECHUNK), jnp.int32),
        pltpu.VMEM((ECHUNK,), jnp.int32),
        pltpu.VMEM((ECHUNK,), jnp.int32),
        pltpu.VMEM((ECHUNK, D), jnp.float32),
        pltpu.VMEM((ECHUNK, D), jnp.float32),
        pltpu.VMEM((NPAD,), jnp.float32),
        pltpu.VMEM((SBE,), jnp.float32),
        pltpu.VMEM((SBE,), jnp.float32),
        pltpu.VMEM((SBE,), jnp.float32),
        pltpu.SemaphoreType.DMA,
        pltpu.SemaphoreType.DMA,
    ]

    @functools.partial(pl.kernel, out_type=out_type, mesh=_MESH,
                       compiler_params=_CP, scratch_types=scratch)
    def k(h_hbm, sd_hbm, nrm2_hbm, *refs):
        outs = list(refs[: len(out_type)])
        (sp_h, ebuf, sidx, didx, abuf, bbuf, nbuf,
         dob, eob, npb, sema, semb) = refs[len(out_type):]
        o_dot = outs.pop(0) if want_dot else None
        o_eud = outs.pop(0) if want_eud else None
        o_np = outs.pop(0) if want_np else None

        cid = lax.axis_index("c")
        sid = lax.axis_index("s")
        wid = cid * 16 + sid

        pltpu.sync_copy(h_hbm.at[pl.ds(sid * ROWS_PER_TILE, ROWS_PER_TILE)],
                        sp_h.at[pl.ds(sid * ROWS_PER_TILE, ROWS_PER_TILE)])
        if want_np:
            pltpu.sync_copy(nrm2_hbm, nbuf)
        plsc.subcore_barrier()

        tile_c0 = wid * n_chunks

        @pl.loop(0, n_sb)
        def _(sb):
            pltpu.sync_copy(sd_hbm.at[pl.ds(tile_c0 + sb * ESB, ESB)], ebuf)

            @pl.loop(0, ESB)
            def _(lc):
                @pl.loop(0, ECHUNK, step=16)
                def _(j):
                    sidx[pl.ds(j, 16)] = ebuf[lc, pl.ds(j, 16)]
                    didx[pl.ds(j, 16)] = ebuf[lc, pl.ds(ECHUNK + j, 16)]
                pltpu.async_copy(sp_h.at[sidx], abuf, sema)
                pltpu.async_copy(sp_h.at[didx], bbuf, semb)
                pltpu.make_async_copy(sp_h.at[sidx], abuf, sema).wait()
                pltpu.make_async_copy(sp_h.at[didx], bbuf, semb).wait()

                @pl.loop(0, ECHUNK // 16)
                def _(g):
                    rows16 = g * 16 + _IOTA16()

                    def jbody(j, carry):
                        dacc, eacc = carry
                        jv = jnp.full((16,), 0, jnp.int32) + j
                        av = plsc.load_gather(abuf, [rows16, jv])
                        bv = plsc.load_gather(bbuf, [rows16, jv])
                        if want_dot:
                            dacc = dacc + av * bv
                        if want_eud:
                            dd = av - bv + jnp.float32(1e-6)
                            eacc = eacc + dd * dd
                        return dacc, eacc

                    z = jnp.zeros((16,), jnp.float32)
                    dacc, eacc = pl.loop(0, D, init_carry=(z, z),
                                         unroll=8)(jbody)
                    o16 = lc * ECHUNK + g * 16
                    if want_dot:
                        dob[pl.ds(o16, 16)] = dacc
                    if want_eud:
                        eob[pl.ds(o16, 16)] = eacc
                    if want_np:
                        sv = sidx[pl.ds(g * 16, 16)]
                        dv = didx[pl.ds(g * 16, 16)]
                        na = plsc.load_gather(nbuf, [sv])
                        nb = plsc.load_gather(nbuf, [dv])
                        npb[pl.ds(o16, 16)] = na * nb

            sbe0 = (tile_c0 + sb * ESB) * ECHUNK
            if want_dot:
                pltpu.sync_copy(dob, o_dot.at[pl.ds(sbe0, SBE)])
            if want_eud:
                pltpu.sync_copy(eob, o_eud.at[pl.ds(sbe0, SBE)])
            if want_np:
                pltpu.sync_copy(npb, o_np.at[pl.ds(sbe0, SBE)])

    return k(h, sd_e, nrm2)


def _sc_degree(ws, dst2d):
    n_w = len(ws)
    n_chunks = EPAD // (N_TILES * CHUNK)
    out_type = [jax.ShapeDtypeStruct((N_TILES, NPAD), jnp.float32)
                for _ in range(n_w)]
    scratch = (
        [pltpu.VMEM((NPAD,), jnp.float32) for _ in range(n_w)]
        + [pltpu.VMEM((1, CHUNK), jnp.int32)]
        + [pltpu.VMEM((CHUNK,), jnp.float32) for _ in range(n_w)]
    )

    @functools.partial(pl.kernel, out_type=out_type, mesh=_MESH,
                       compiler_params=_CP, scratch_types=scratch)
    def k(*refs):
        w_hbm = refs[:n_w]
        d_hbm = refs[n_w]
        outs = refs[n_w + 1: 2 * n_w + 1]
        degb = refs[2 * n_w + 1: 3 * n_w + 1]
        didx = refs[3 * n_w + 1]
        wbuf = refs[3 * n_w + 2: 4 * n_w + 2]

        cid = lax.axis_index("c")
        sid = lax.axis_index("s")
        wid = cid * 16 + sid

        @pl.loop(0, NPAD, step=16)
        def _(i):
            for b in range(n_w):
                degb[b][pl.ds(i, 16)] = jnp.zeros((16,), jnp.float32)

        tile_row0 = wid * n_chunks

        @pl.loop(0, n_chunks)
        def _(ci):
            row = tile_row0 + ci
            pltpu.sync_copy(d_hbm.at[row], didx)
            for b in range(n_w):
                pltpu.sync_copy(w_hbm[b].at[pl.ds(row * CHUNK, CHUNK)], wbuf[b])

            @pl.loop(0, CHUNK // 16)
            def _(g):
                iv = didx[0, pl.ds(g * 16, 16)]
                for b in range(n_w):
                    vv = wbuf[b][pl.ds(g * 16, 16)]
                    plsc.addupdate_scatter(degb[b], [iv], vv)

        for b in range(n_w):
            pltpu.sync_copy(degb[b], outs[b].at[wid])

    return k(*ws, dst2d)


MCHUNK = 128
MSB = 16


def _sc_messages(hmd_pad, edata):
    n_chunks = EPAD // (N_TILES * MCHUNK)
    n_sb = n_chunks // MSB

    out_type = jax.ShapeDtypeStruct((2 * NSTAGE, D), jnp.float32)
    scratch = (
        [pltpu.VMEM_SHARED((NSTAGE, D), jnp.float32)]
        + [pltpu.VMEM((MCHUNK, D), jnp.float32)] * 2
        + [pltpu.VMEM((MSB, 384), jnp.int32)]
        + [pltpu.VMEM((MCHUNK,), jnp.int32)] * 4
        + [pltpu.SemaphoreType.DMA] * 2
    )

    @functools.partial(pl.kernel, out_type=out_type, mesh=_MESH,
                       compiler_params=_CP, scratch_types=scratch)
    def k(hm_hbm, ed_hbm, out_hbm,
          sp_out, gb0, gb1, ebuf, si0, si1, di0, di1, gs0, gs1):
        cid = lax.axis_index("c")
        sid = lax.axis_index("s")
        wid = cid * 16 + sid
        gb = [gb0, gb1]
        si = [si0, si1]
        di = [di0, di1]
        gsem = [gs0, gs1]

        @pl.loop(0, MCHUNK)
        def _(r):
            @pl.loop(0, D, step=16)
            def _(j):
                gb0.at[r][pl.ds(j, 16)] = jnp.zeros((16,), jnp.float32)

        r0 = sid * ROWS_PER_TILE
        @pl.loop(0, ROWS_PER_TILE, step=MCHUNK)
        def _(rr):
            pltpu.sync_copy(gb0, sp_out.at[pl.ds(r0 + rr, MCHUNK)])
        plsc.subcore_barrier()

        tile_c0 = wid * n_chunks

        def build_idx(lc, p):
            @pl.loop(0, MCHUNK, step=16)
            def _(j):
                si[p][pl.ds(j, 16)] = ebuf[lc, pl.ds(j, 16)]
                di[p][pl.ds(j, 16)] = ebuf[lc, pl.ds(MCHUNK + j, 16)]

        def fire_gather(p):
            pltpu.async_copy(hm_hbm.at[si[p]], gb[p], gsem[p])

        def wait_gather(p):
            pltpu.make_async_copy(hm_hbm.at[si[p]], gb[p], gsem[p]).wait()

        def compute_scatter(lc, p):
            @pl.loop(0, MCHUNK // 16)
            def _(g):
                rows16 = g * 16 + _IOTA16()
                wv = plsc.bitcast(ebuf[lc, pl.ds(2 * MCHUNK + g * 16, 16)],
                                  jnp.float32)

                def jbody(j, _):
                    jv = jnp.full((16,), 0, jnp.int32) + j
                    col = plsc.load_gather(gb[p], [rows16, jv])
                    plsc.store_scatter(gb[p], [rows16, jv], col * wv)
                    return 0

                pl.loop(0, D, init_carry=0, unroll=8)(jbody)

            pltpu.sync_copy(gb[p], sp_out.at[di[p]], add=True)

        @pl.loop(0, n_sb)
        def _(sb):
            pltpu.sync_copy(ed_hbm.at[pl.ds(tile_c0 + sb * MSB, MSB)], ebuf)
            build_idx(0, 0)
            fire_gather(0)

            @pl.loop(0, MSB, step=2)
            def _(lc):
                build_idx(lc + 1, 1)
                wait_gather(0)
                fire_gather(1)
                compute_scatter(lc, 0)

                @pl.when(lc + 2 < MSB)
                def _():
                    build_idx(lc + 2, 0)
                    fire_gather(0)
                wait_gather(1)
                compute_scatter(lc + 1, 1)

        plsc.subcore_barrier()
        for c in range(2):
            @pl.when(cid == c)
            def _():
                pltpu.sync_copy(
                    sp_out.at[pl.ds(r0, ROWS_PER_TILE)],
                    out_hbm.at[pl.ds(c * NSTAGE + r0, ROWS_PER_TILE)])

    return k(hmd_pad, edata)


_NB = 10
_RB = N // _NB


def _tc_matmul_stats(h, W, with_stats):
    def body(h_ref, w_ref, hm_ref, *stat_ref):
        hb = h_ref[...]
        hm_ref[...] = jnp.dot(hb, w_ref[...],
                              preferred_element_type=jnp.float32)
        if with_stats:
            stat_ref[0][...] = jnp.broadcast_to(
                jnp.sum(hb * hb, axis=1, keepdims=True), hb.shape)

    out_shape = [jax.ShapeDtypeStruct((N, D), jnp.float32)]
    out_specs = [pl.BlockSpec((_RB, D), lambda i: (i, 0))]
    if with_stats:
        out_shape.append(jax.ShapeDtypeStruct((N, D), jnp.float32))
        out_specs.append(pl.BlockSpec((_RB, D), lambda i: (i, 0)))

    res = pl.pallas_call(
        body,
        grid=(_NB,),
        in_specs=[pl.BlockSpec((_RB, D), lambda i: (i, 0)),
                  pl.BlockSpec((D, D), lambda i: (0, 0))],
        out_specs=out_specs,
        out_shape=out_shape,
    )(h, W)
    if with_stats:
        hm, st = res
        nrm2 = jnp.concatenate(
            [st[:, 0], jnp.zeros((NPAD - N,), jnp.float32)])
        return hm, nrm2
    return res[0]


_EB = EPAD // 128 // 10
_E_ROWS = E // 128


def _tc_edge_weights(dot, eud, nprod, mode):
    want_cos = mode in ("both", "cos")
    want_eud = mode in ("both", "eud")

    def body(*refs):
        i = 0
        dot_r = eud_r = np_r = None
        if want_cos:
            dot_r = refs[i]; i += 1
        if want_eud:
            eud_r = refs[i]; i += 1
        if want_cos:
            np_r = refs[i]; i += 1
        outs = refs[i:]
        pid = pl.program_id(0)
        row0 = pid * _EB
        rows = row0 + lax.broadcasted_iota(jnp.int32, (_EB, 128), 0)
        valid = rows < _E_ROWS
        oi = 0
        if want_cos:
            wc = dot_r[...] / jnp.maximum(jnp.sqrt(np_r[...]),
                                          jnp.float32(1e-8))
            outs[oi][...] = jnp.where(valid, wc, 0.0)
            oi += 1
        if want_eud:
            we = jnp.sqrt(jnp.maximum(eud_r[...], 0.0))
            outs[oi][...] = jnp.where(valid, we, 0.0)

    ins, in_specs = [], []
    spec = pl.BlockSpec((_EB, 128), lambda i: (i, 0))
    if want_cos:
        ins.append(dot.reshape(-1, 128)); in_specs.append(spec)
    if want_eud:
        ins.append(eud.reshape(-1, 128)); in_specs.append(spec)
    if want_cos:
        ins.append(nprod.reshape(-1, 128)); in_specs.append(spec)
    n_out = int(want_cos) + int(want_eud)
    res = pl.pallas_call(
        body,
        grid=(10,),
        in_specs=in_specs,
        out_specs=[spec] * n_out,
        out_shape=[jax.ShapeDtypeStruct((EPAD // 128, 128), jnp.float32)] * n_out,
    )(*ins)
    return [r.reshape(-1) for r in res]


def _tc_dinv(parts):
    def body(p_ref, o_ref):
        s = jnp.sum(p_ref[...], axis=0, keepdims=True) + 1.0
        safe = jnp.where(s > 0, s, 1.0)
        dinv = jnp.where(s > 0, 1.0 / jnp.sqrt(safe), 0.0)
        o_ref[...] = jnp.broadcast_to(dinv, (8, p_ref.shape[1]))

    res = pl.pallas_call(
        body,
        grid=(8,),
        in_specs=[pl.BlockSpec((N_TILES, NPAD // 8), lambda i: (0, i))],
        out_specs=pl.BlockSpec((8, NPAD // 8), lambda i: (0, i)),
        out_shape=jax.ShapeDtypeStruct((8, NPAD), jnp.float32),
    )(parts)
    return res[0]


def _tc_scale_rows(hm, dinv):
    def body(hm_ref, di_ref, o_ref):
        o_ref[...] = hm_ref[...] * di_ref[...]

    blk = pl.BlockSpec((_RB, D), lambda i: (i, 0))
    return pl.pallas_call(
        body,
        grid=(_NB,),
        in_specs=[blk, pl.BlockSpec((_RB, 1), lambda i: (i, 0))],
        out_specs=blk,
        out_shape=jax.ShapeDtypeStruct((N, D), jnp.float32),
    )(hm, dinv[:N].reshape(N, 1))


def _tc_post(agg0, agg1, hmd, dinv, b, relu):
    def body(a0_ref, a1_ref, hmd_ref, di_ref, b_ref, o_ref):
        out = ((a0_ref[...] + a1_ref[...] + hmd_ref[...]) * di_ref[...]
               + b_ref[...])
        if relu:
            out = jnp.maximum(out, 0.0)
        o_ref[...] = out

    blk = pl.BlockSpec((_RB, D), lambda i: (i, 0))
    res = pl.pallas_call(
        body,
        grid=(_NB,),
        in_specs=[blk, blk, blk,
                  pl.BlockSpec((_RB, 1), lambda i: (i, 0)),
                  pl.BlockSpec((1, D), lambda i: (0, 0))],
        out_specs=blk,
        out_shape=jax.ShapeDtypeStruct((N, D), jnp.float32),
    )(agg0, agg1, hmd, dinv[:N].reshape(N, 1), b.reshape(1, D))
    return res


def _tc_attention(x1, x2, aw1, ab1, aw2):
    def body(x1_ref, x2_ref, w1_ref, b1_ref, w2_ref, o_ref):
        x1b, x2b = x1_ref[...], x2_ref[...]
        t1 = jnp.tanh(jnp.dot(x1b, w1_ref[...],
                              preferred_element_type=jnp.float32) + b1_ref[...])
        t2 = jnp.tanh(jnp.dot(x2b, w1_ref[...],
                              preferred_element_type=jnp.float32) + b1_ref[...])
        s1 = jnp.dot(t1, w2_ref[...], preferred_element_type=jnp.float32)
        s2 = jnp.dot(t2, w2_ref[...], preferred_element_type=jnp.float32)
        m = jnp.maximum(s1, s2)
        e1 = jnp.exp(s1 - m)
        e2 = jnp.exp(s2 - m)
        o_ref[...] = (e1 * x1b + e2 * x2b) / (e1 + e2)

    blk = pl.BlockSpec((_RB, D), lambda i: (i, 0))
    res = pl.pallas_call(
        body,
        grid=(_NB,),
        in_specs=[blk, blk,
                  pl.BlockSpec((D, 64), lambda i: (0, 0)),
                  pl.BlockSpec((1, 64), lambda i: (0, 0)),
                  pl.BlockSpec((64, 1), lambda i: (0, 0))],
        out_specs=blk,
        out_shape=jax.ShapeDtypeStruct((N, D), jnp.float32),
    )(x1, x2, aw1, ab1.reshape(1, 64), aw2)
    return res


def _pad_rows(a):
    return jnp.pad(a, ((0, NSTAGE - N), (0, 0)))


def _conv(h_hm, w_list, sd_m, dst2d, bias, relu):
    deg_parts = _sc_degree(w_list, dst2d)
    dinvs = [_tc_dinv(p) for p in deg_parts]
    outs = []
    for b in range(len(w_list)):
        hmd = _tc_scale_rows(h_hm, dinvs[b])
        wm = lax.bitcast_convert_type(w_list[b], jnp.int32).reshape(-1, 128)
        edata = jnp.concatenate([sd_m, wm], axis=1)
        agg = _sc_messages(_pad_rows(hmd), edata)
        outs.append(_tc_post(agg[:N], agg[NSTAGE:NSTAGE + N], hmd,
                             dinvs[b], bias, relu))
    return outs


def kernel(x, edge_index, W1, b1, W2, b2, att_w1, att_b1, att_w2):
    src = edge_index[0]
    dst = edge_index[1]
    pad = EPAD - E
    src1d = jnp.pad(src, (0, pad))
    dst1d = jnp.pad(dst, (0, pad))
    dst2d = dst1d.reshape(EPAD // CHUNK, 1, CHUNK)
    sd_e = jnp.concatenate([src1d.reshape(-1, ECHUNK),
                            dst1d.reshape(-1, ECHUNK)], axis=1)
    sd_m = jnp.concatenate([src1d.reshape(-1, 128),
                            dst1d.reshape(-1, 128)], axis=1)

    hm1, nrm2x = _tc_matmul_stats(x, W1, with_stats=True)
    dot, eud, nprod = _sc_edge_stats(_pad_rows(x), sd_e, nrm2x, "both")
    w_cos, w_eud = _tc_edge_weights(dot, eud, nprod, "both")
    x1, x2 = _conv(hm1, [w_cos, w_eud], sd_m, dst2d, b1, relu=True)

    hm2a, nrm2x1 = _tc_matmul_stats(x1, W2, with_stats=True)
    dot1, nprod1 = _sc_edge_stats(_pad_rows(x1), sd_e, nrm2x1, "cos")
    (w1c,) = _tc_edge_weights(dot1, None, nprod1, "cos")
    (x1o,) = _conv(hm2a, [w1c], sd_m, dst2d, b2, relu=False)

    hm2b = _tc_matmul_stats(x2, W2, with_stats=False)
    zn = jnp.zeros((NPAD,), jnp.float32)
    (eud2,) = _sc_edge_stats(_pad_rows(x2), sd_e, zn, "eud")
    (w2e,) = _tc_edge_weights(None, eud2, None, "eud")
    (x2o,) = _conv(hm2b, [w2e], sd_m, dst2d, b2, relu=False)

    return _tc_attention(x1o, x2o, att_w1, att_b1, att_w2)
